# baseline (speedup 1.0000x reference)
import jax
import jax.numpy as jnp
from jax.experimental import pallas as pl
from jax.experimental.pallas import tpu as pltpu


def _router_kernel(x_ref, wr_ref, br_ref, gate_ref, xbf_ref):
    xb = x_ref[...].astype(jnp.bfloat16)
    xbf_ref[...] = xb
    logits = jnp.dot(xb, wr_ref[...], preferred_element_type=jnp.float32)
    logits = logits + br_ref[...]
    m = jnp.max(logits, axis=1, keepdims=True)
    p = jnp.exp(logits - m)
    gate_ref[...] = p / jnp.sum(p, axis=1, keepdims=True)


def _moe_kernel(x_ref, gate_ref, we_ref, be_ref, out_ref):
    e2 = pl.program_id(2)

    gate = gate_ref[...]
    lane = jax.lax.broadcasted_iota(jnp.int32, gate.shape, 1)
    g0 = jnp.sum(
        jnp.where(lane == 2 * e2, gate, 0.0), axis=1, keepdims=True
    )
    g1 = jnp.sum(
        jnp.where(lane == 2 * e2 + 1, gate, 0.0), axis=1, keepdims=True
    )

    x = x_ref[...]
    y0 = jnp.dot(x, we_ref[0], preferred_element_type=jnp.float32)
    contrib0 = g0 * (y0 + be_ref[0])

    @pl.when(e2 == 0)
    def _init():
        out_ref[...] = contrib0

    @pl.when(e2 != 0)
    def _accum():
        out_ref[...] += contrib0

    y1 = jnp.dot(x, we_ref[1], preferred_element_type=jnp.float32)
    out_ref[...] += g1 * (y1 + be_ref[1])


def kernel(x, Wr, br, We, be):
    T, D = x.shape
    E = Wr.shape[1]

    tm_r = 2048
    gate, x_bf = pl.pallas_call(
        _router_kernel,
        grid=(T // tm_r,),
        in_specs=[
            pl.BlockSpec((tm_r, D), lambda t: (t, 0)),
            pl.BlockSpec((D, E), lambda t: (0, 0)),
            pl.BlockSpec((1, E), lambda t: (0, 0)),
        ],
        out_specs=[
            pl.BlockSpec((tm_r, E), lambda t: (t, 0)),
            pl.BlockSpec((tm_r, D), lambda t: (t, 0)),
        ],
        out_shape=[
            jax.ShapeDtypeStruct((T, E), jnp.float32),
            jax.ShapeDtypeStruct((T, D), jnp.bfloat16),
        ],
    )(x, Wr.astype(jnp.bfloat16), br.reshape(1, E))

    tm = 2048
    nn = 512
    grid = (T // tm, D // nn, E // 2)
    out = pl.pallas_call(
        _moe_kernel,
        grid=grid,
        in_specs=[
            pl.BlockSpec((tm, D), lambda t, n, e: (t, 0)),
            pl.BlockSpec((tm, E), lambda t, n, e: (t, 0)),
            pl.BlockSpec((2, D, nn), lambda t, n, e: (e, 0, n)),
            pl.BlockSpec((2, 1, nn), lambda t, n, e: (e, 0, n)),
        ],
        out_specs=pl.BlockSpec((tm, nn), lambda t, n, e: (t, n)),
        out_shape=jax.ShapeDtypeStruct((T, D), jnp.float32),
        compiler_params=pltpu.CompilerParams(
            dimension_semantics=("parallel", "parallel", "arbitrary")
        ),
    )(x_bf, gate, We.astype(jnp.bfloat16), be.reshape(E, 1, D))
    return out


# speedup vs baseline: 1.0749x; 1.0749x over previous
import jax
import jax.numpy as jnp
from jax.experimental import pallas as pl
from jax.experimental.pallas import tpu as pltpu


def _router_kernel(x_ref, wr_ref, br_ref, gate_ref, xbf_ref):
    xb = x_ref[...].astype(jnp.bfloat16)
    xbf_ref[...] = xb
    logits = jnp.dot(xb, wr_ref[...], preferred_element_type=jnp.float32)
    logits = logits + br_ref[...]
    m = jnp.max(logits, axis=1, keepdims=True)
    p = jnp.exp(logits - m)
    gate_ref[...] = p / jnp.sum(p, axis=1, keepdims=True)


def _moe_kernel(x_ref, gate_ref, we_ref, be_ref, out_ref):
    x = x_ref[...]
    gate = gate_ref[...]
    lane = jax.lax.broadcasted_iota(jnp.int32, gate.shape, 1)
    n_e = we_ref.shape[0]
    for e in range(n_e):
        g = jnp.sum(
            jnp.where(lane == e, gate, 0.0), axis=1, keepdims=True
        )
        y = jnp.dot(x, we_ref[e], preferred_element_type=jnp.float32)
        contrib = g * (y + be_ref[e])
        if e == 0:
            out_ref[...] = contrib
        else:
            out_ref[...] += contrib


def kernel(x, Wr, br, We, be):
    T, D = x.shape
    E = Wr.shape[1]

    tm_r = 2048
    gate, x_bf = pl.pallas_call(
        _router_kernel,
        grid=(T // tm_r,),
        in_specs=[
            pl.BlockSpec((tm_r, D), lambda t: (t, 0)),
            pl.BlockSpec((D, E), lambda t: (0, 0)),
            pl.BlockSpec((1, E), lambda t: (0, 0)),
        ],
        out_specs=[
            pl.BlockSpec((tm_r, E), lambda t: (t, 0)),
            pl.BlockSpec((tm_r, D), lambda t: (t, 0)),
        ],
        out_shape=[
            jax.ShapeDtypeStruct((T, E), jnp.float32),
            jax.ShapeDtypeStruct((T, D), jnp.bfloat16),
        ],
    )(x, Wr.astype(jnp.bfloat16), br.reshape(1, E))

    tm = 1024
    nn = 512
    grid = (T // tm, D // nn)
    out = pl.pallas_call(
        _moe_kernel,
        grid=grid,
        in_specs=[
            pl.BlockSpec((tm, D), lambda t, n: (t, 0)),
            pl.BlockSpec((tm, E), lambda t, n: (t, 0)),
            pl.BlockSpec((E, D, nn), lambda t, n: (0, 0, n)),
            pl.BlockSpec((E, 1, nn), lambda t, n: (0, 0, n)),
        ],
        out_specs=pl.BlockSpec((tm, nn), lambda t, n: (t, n)),
        out_shape=jax.ShapeDtypeStruct((T, D), jnp.float32),
        compiler_params=pltpu.CompilerParams(
            dimension_semantics=("parallel", "parallel")
        ),
    )(x_bf, gate, We.astype(jnp.bfloat16), be.reshape(E, 1, D))
    return out


# speedup vs baseline: 1.1302x; 1.0515x over previous
import jax
import jax.numpy as jnp
from jax.experimental import pallas as pl
from jax.experimental.pallas import tpu as pltpu


def _router_kernel(x_ref, wr_ref, br_ref, gate_ref):
    logits = jnp.dot(
        x_ref[...], wr_ref[...], preferred_element_type=jnp.float32
    )
    logits = logits + br_ref[...]
    m = jnp.max(logits, axis=1, keepdims=True)
    p = jnp.exp(logits - m)
    gate_ref[...] = p / jnp.sum(p, axis=1, keepdims=True)


def _moe_kernel(x_ref, gate_ref, we_ref, be_ref, out_ref):
    x = x_ref[...]
    gate = gate_ref[...]
    lane = jax.lax.broadcasted_iota(jnp.int32, gate.shape, 1)
    n_e = we_ref.shape[0]
    for e in range(n_e):
        g = jnp.sum(
            jnp.where(lane == e, gate, 0.0), axis=1, keepdims=True
        )
        y = jnp.dot(x, we_ref[e], preferred_element_type=jnp.float32)
        contrib = g * (y + be_ref[e])
        if e == 0:
            out_ref[...] = contrib
        else:
            out_ref[...] += contrib


def kernel(x, Wr, br, We, be):
    T, D = x.shape
    E = Wr.shape[1]

    tm_r = 2048
    gate = pl.pallas_call(
        _router_kernel,
        grid=(T // tm_r,),
        in_specs=[
            pl.BlockSpec((tm_r, D), lambda t: (t, 0)),
            pl.BlockSpec((D, E), lambda t: (0, 0)),
            pl.BlockSpec((1, E), lambda t: (0, 0)),
        ],
        out_specs=pl.BlockSpec((tm_r, E), lambda t: (t, 0)),
        out_shape=jax.ShapeDtypeStruct((T, E), jnp.float32),
    )(x, Wr, br.reshape(1, E))

    tm = 1024
    nn = 256
    grid = (T // tm, D // nn)
    out = pl.pallas_call(
        _moe_kernel,
        grid=grid,
        in_specs=[
            pl.BlockSpec((tm, D), lambda t, n: (t, 0)),
            pl.BlockSpec((tm, E), lambda t, n: (t, 0)),
            pl.BlockSpec((E, D, nn), lambda t, n: (0, 0, n)),
            pl.BlockSpec((E, 1, nn), lambda t, n: (0, 0, n)),
        ],
        out_specs=pl.BlockSpec((tm, nn), lambda t, n: (t, n)),
        out_shape=jax.ShapeDtypeStruct((T, D), jnp.float32),
        compiler_params=pltpu.CompilerParams(
            dimension_semantics=("parallel", "parallel")
        ),
    )(x, gate, We, be.reshape(E, 1, D))
    return out
